# Initial kernel scaffold; baseline (speedup 1.0000x reference)
import jax
import jax.numpy as jnp
from jax.experimental import pallas as pl


def kernel(x, edge_index, edge_label_index, H, W_xz, b_xz, W_hz, b_hz, W_xr, b_xr, W_hr, b_hr, W_xh, b_xh, W_hh, b_hh, W_post, b_post):
    raise NotImplementedError("write your pallas kernel here")


# speedup vs baseline: 9.0344x; 9.0344x over previous
import functools

import jax
import jax.numpy as jnp
from jax import lax
from jax.experimental import pallas as pl
from jax.experimental.pallas import tpu as pltpu
from jax.experimental.pallas import tpu_sc as plsc

N = 10000
E = 320000
EL = 100000
D = 128

NC = 2
NS = 16
CB = 128
NCHUNKS_E = E // CB
N_PAD = 10240
ROWS_PER_TILE = N_PAD // NS

_MESH = plsc.VectorSubcoreMesh(
    core_axis_name="c", subcore_axis_name="s", num_cores=NC, num_subcores=NS)

f32 = jnp.float32
i32 = jnp.int32


@functools.partial(
    pl.kernel,
    out_type=(jax.ShapeDtypeStruct((N_PAD, D), f32),
              jax.ShapeDtypeStruct((N_PAD, D), f32)),
    mesh=_MESH,
    scratch_types=[
        pltpu.VMEM((CB,), i32),
        pltpu.VMEM((CB, D), f32),
        pltpu.VMEM_SHARED((N_PAD, D), f32),
        pltpu.SemaphoreType.DMA,
    ],
)
def _deg_kernel(src_hbm, ones_hbm, zeros_hbm, deg0_hbm, deg1_hbm,
                idx_v, ones_v, acc_sh, sem):
    c = lax.axis_index("c")
    s = lax.axis_index("s")
    wid = c * NS + s

    pltpu.sync_copy(zeros_hbm.at[pl.ds(0, ROWS_PER_TILE)],
                    acc_sh.at[pl.ds(s * ROWS_PER_TILE, ROWS_PER_TILE)])
    pltpu.sync_copy(ones_hbm, ones_v)
    plsc.subcore_barrier()

    cnt = (NCHUNKS_E - wid + (NC * NS - 1)) // (NC * NS)

    def body(i, _):
        base = (wid + i * NC * NS) * CB
        pltpu.sync_copy(src_hbm.at[pl.ds(base, CB)], idx_v)
        pltpu.sync_copy(ones_v, acc_sh.at[idx_v], add=True)
        return 0
    lax.fori_loop(0, cnt, body, 0)
    plsc.subcore_barrier()

    @pl.when(c == 0)
    def _():
        pltpu.sync_copy(acc_sh.at[pl.ds(s * ROWS_PER_TILE, ROWS_PER_TILE)],
                        deg0_hbm.at[pl.ds(s * ROWS_PER_TILE, ROWS_PER_TILE)])

    @pl.when(c == 1)
    def _():
        pltpu.sync_copy(acc_sh.at[pl.ds(s * ROWS_PER_TILE, ROWS_PER_TILE)],
                        deg1_hbm.at[pl.ds(s * ROWS_PER_TILE, ROWS_PER_TILE)])


@functools.partial(
    pl.kernel,
    out_type=(jax.ShapeDtypeStruct((N_PAD, D), f32),
              jax.ShapeDtypeStruct((N_PAD, D), f32)),
    mesh=_MESH,
    scratch_types=[
        pltpu.VMEM((CB,), i32),
        pltpu.VMEM((CB,), i32),
        pltpu.VMEM((CB, D), f32),
        pltpu.VMEM_SHARED((N_PAD, D), f32),
        pltpu.SemaphoreType.DMA,
    ],
)
def _prop2_kernel(xs_hbm, hs_hbm, src_hbm, dst_hbm, zeros_hbm,
                  outx_hbm, outh_hbm, idxs_v, idxd_v, rows_v, acc_sh, sem):
    c = lax.axis_index("c")
    s = lax.axis_index("s")

    pltpu.sync_copy(zeros_hbm,
                    acc_sh.at[pl.ds(s * ROWS_PER_TILE, ROWS_PER_TILE)])
    plsc.subcore_barrier()

    def run(table_hbm):
        def body(i, _):
            base = (s + i * NS) * CB
            pltpu.sync_copy(src_hbm.at[pl.ds(base, CB)], idxs_v)
            pltpu.sync_copy(dst_hbm.at[pl.ds(base, CB)], idxd_v)
            pltpu.async_copy(table_hbm.at[idxs_v], rows_v, sem).wait()
            pltpu.sync_copy(rows_v, acc_sh.at[idxd_v], add=True)
            return 0
        lax.fori_loop(0, (NCHUNKS_E - s + (NS - 1)) // NS, body, 0)

    @pl.when(c == 0)
    def _():
        run(xs_hbm)

    @pl.when(c == 1)
    def _():
        run(hs_hbm)

    plsc.subcore_barrier()

    @pl.when(c == 0)
    def _():
        pltpu.sync_copy(acc_sh.at[pl.ds(s * ROWS_PER_TILE, ROWS_PER_TILE)],
                        outx_hbm.at[pl.ds(s * ROWS_PER_TILE, ROWS_PER_TILE)])

    @pl.when(c == 1)
    def _():
        pltpu.sync_copy(acc_sh.at[pl.ds(s * ROWS_PER_TILE, ROWS_PER_TILE)],
                        outh_hbm.at[pl.ds(s * ROWS_PER_TILE, ROWS_PER_TILE)])


@functools.partial(
    pl.kernel,
    out_type=(jax.ShapeDtypeStruct((N_PAD, D), f32),
              jax.ShapeDtypeStruct((N_PAD, D), f32)),
    mesh=_MESH,
    scratch_types=[
        pltpu.VMEM((CB,), i32),
        pltpu.VMEM((CB,), i32),
        pltpu.VMEM((CB, D), f32),
        pltpu.VMEM_SHARED((N_PAD, D), f32),
        pltpu.SemaphoreType.DMA,
    ],
)
def _prop1_kernel(gs_hbm, src_hbm, dst_hbm, zeros_hbm,
                  out0_hbm, out1_hbm, idxs_v, idxd_v, rows_v, acc_sh, sem):
    c = lax.axis_index("c")
    s = lax.axis_index("s")
    wid = c * NS + s

    pltpu.sync_copy(zeros_hbm,
                    acc_sh.at[pl.ds(s * ROWS_PER_TILE, ROWS_PER_TILE)])
    plsc.subcore_barrier()

    def body(i, _):
        base = (wid + i * NC * NS) * CB
        pltpu.sync_copy(src_hbm.at[pl.ds(base, CB)], idxs_v)
        pltpu.sync_copy(dst_hbm.at[pl.ds(base, CB)], idxd_v)
        pltpu.async_copy(gs_hbm.at[idxs_v], rows_v, sem).wait()
        pltpu.sync_copy(rows_v, acc_sh.at[idxd_v], add=True)
        return 0
    lax.fori_loop(0, (NCHUNKS_E - wid + (NC * NS - 1)) // (NC * NS), body, 0)
    plsc.subcore_barrier()

    @pl.when(c == 0)
    def _():
        pltpu.sync_copy(acc_sh.at[pl.ds(s * ROWS_PER_TILE, ROWS_PER_TILE)],
                        out0_hbm.at[pl.ds(s * ROWS_PER_TILE, ROWS_PER_TILE)])

    @pl.when(c == 1)
    def _():
        pltpu.sync_copy(acc_sh.at[pl.ds(s * ROWS_PER_TILE, ROWS_PER_TILE)],
                        out1_hbm.at[pl.ds(s * ROWS_PER_TILE, ROWS_PER_TILE)])


NCHUNKS_L = (EL + CB - 1) // CB
LAST_START = EL - CB


@functools.partial(
    pl.kernel,
    out_type=jax.ShapeDtypeStruct((EL, 16), f32),
    mesh=_MESH,
    scratch_types=[
        pltpu.VMEM((CB,), i32),
        pltpu.VMEM((CB,), i32),
        pltpu.VMEM((CB, D), f32),
        pltpu.VMEM((CB, D), f32),
        pltpu.VMEM((CB, 16), f32),
        pltpu.SemaphoreType.DMA,
    ],
)
def _pred_kernel(h_hbm, hw_hbm, srcl_hbm, dstl_hbm, part_hbm,
                 idxa_v, idxb_v, ra_v, rb_v, part_v, sem):
    c = lax.axis_index("c")
    s = lax.axis_index("s")
    wid = c * NS + s

    cnt = (NCHUNKS_L - wid + (NC * NS - 1)) // (NC * NS)

    def body(i, _):
        chunk = wid + i * NC * NS
        start = pl.multiple_of(jnp.minimum(chunk * CB, LAST_START), 32)
        pltpu.sync_copy(srcl_hbm.at[pl.ds(start, CB)], idxa_v)
        pltpu.sync_copy(dstl_hbm.at[pl.ds(start, CB)], idxb_v)
        pltpu.async_copy(h_hbm.at[idxa_v], ra_v, sem).wait()
        pltpu.async_copy(hw_hbm.at[idxb_v], rb_v, sem).wait()

        def ebody(e, _):
            acc = ra_v[e, pl.ds(0, 16)] * rb_v[e, pl.ds(0, 16)]
            for k in range(1, D // 16):
                acc = acc + (ra_v[e, pl.ds(k * 16, 16)]
                             * rb_v[e, pl.ds(k * 16, 16)])
            part_v[e, :] = acc
            return 0
        lax.fori_loop(0, CB, ebody, 0)
        pltpu.sync_copy(part_v, part_hbm.at[pl.ds(start, CB)])
        return 0
    lax.fori_loop(0, cnt, body, 0)


def _reduce_body(p_ref, s_ref, c_ref, out_ref):
    out_ref[...] = (jnp.dot(p_ref[...], s_ref[...], preferred_element_type=f32)
                    + c_ref[...])


def _stage_h(part2d, smat, csum8):
    nrows = EL * 16 // D
    return pl.pallas_call(
        _reduce_body,
        out_shape=jax.ShapeDtypeStruct((nrows, 8), f32),
    )(part2d, smat, csum8)


RB = 1000
GRID = N // RB


def _row_spec():
    return pl.BlockSpec((RB, D), lambda i: (i, 0))


def _full_spec(shape):
    return pl.BlockSpec(shape, lambda i: tuple(0 for _ in shape))


def _scale_body(x_ref, h_ref, degb_ref, xs_ref, hs_ref, disb_ref):
    deg = degb_ref[...]
    dis = jnp.where(deg > 0, lax.rsqrt(jnp.maximum(deg, 1e-12)), 0.0)
    xs_ref[...] = -(x_ref[...] * dis)
    hs_ref[...] = -(h_ref[...] * dis)
    disb_ref[...] = dis


def _stage_b(x, H, degb):
    return pl.pallas_call(
        _scale_body,
        grid=(GRID,),
        in_specs=[_row_spec()] * 3,
        out_specs=[_row_spec()] * 3,
        out_shape=[jax.ShapeDtypeStruct((N, D), f32)] * 3,
    )(x, H, degb)


def _gates_body(x_ref, h_ref, sxr_ref, shr_ref, disb_ref,
                wzr_ref, bzr_ref, wxh2_ref, whh0_ref, bh_ref,
                z_ref, p_ref, gs_ref):
    dis = disb_ref[...]
    xv = x_ref[...]
    hv = h_ref[...]
    sx = sxr_ref[...] * dis
    sh = shr_ref[...] * dis
    cat = jnp.concatenate([xv, sx, hv, sh], axis=1)
    zr = jnp.dot(cat, wzr_ref[...], preferred_element_type=f32) + bzr_ref[...]
    z = jax.nn.sigmoid(zr[:, :D])
    r = jax.nn.sigmoid(zr[:, D:])
    g = hv * r
    p = (jnp.dot(jnp.concatenate([xv, sx], axis=1), wxh2_ref[...],
                 preferred_element_type=f32)
         + jnp.dot(g, whh0_ref[...], preferred_element_type=f32)
         + bh_ref[...])
    z_ref[...] = z
    p_ref[...] = p
    gs_ref[...] = -(g * dis)


def _stage_d(x, H, sxr, shr, disb, wzr, bzr, wxh2, whh0, bh):
    return pl.pallas_call(
        _gates_body,
        grid=(GRID,),
        in_specs=[_row_spec()] * 5 + [
            _full_spec((4 * D, 2 * D)), _full_spec((1, 2 * D)),
            _full_spec((2 * D, D)), _full_spec((D, D)), _full_spec((1, D)),
        ],
        out_specs=[_row_spec()] * 3,
        out_shape=[jax.ShapeDtypeStruct((N, D), f32)] * 3,
    )(x, H, sxr, shr, disb, wzr, bzr, wxh2, whh0, bh)


def _update_body(z_ref, p_ref, sg0_ref, sg1_ref, disb_ref, h_ref,
                 whh1_ref, wsum_ref, hn_ref, hr_ref, hwr_ref):
    sg = (sg0_ref[...] + sg1_ref[...]) * disb_ref[...]
    ht = jnp.tanh(p_ref[...] + jnp.dot(sg, whh1_ref[...],
                                       preferred_element_type=f32))
    z = z_ref[...]
    hn = z * h_ref[...] + (1.0 - z) * ht
    hrelu = jnp.maximum(hn, 0.0)
    hn_ref[...] = hn
    hr_ref[...] = hrelu
    hwr_ref[...] = hrelu * wsum_ref[...]


def _stage_f(z, p, sg0, sg1, disb, H, whh1, wsum):
    return pl.pallas_call(
        _update_body,
        grid=(GRID,),
        in_specs=[_row_spec()] * 6 + [_full_spec((D, D)), _full_spec((1, D))],
        out_specs=[_row_spec()] * 3,
        out_shape=[jax.ShapeDtypeStruct((N, D), f32)] * 3,
    )(z, p, sg0, sg1, disb, H, whh1, wsum)


def kernel(x, edge_index, edge_label_index, H,
           W_xz, b_xz, W_hz, b_hz, W_xr, b_xr, W_hr, b_hr,
           W_xh, b_xh, W_hh, b_hh, W_post, b_post):
    src = edge_index[0]
    dst = edge_index[1]
    srcl = edge_label_index[0]
    dstl = edge_label_index[1]

    zeros128 = jnp.zeros((ROWS_PER_TILE, D), f32)

    ones128 = jnp.ones((CB, D), f32)
    deg0, deg1 = _deg_kernel(src, ones128, zeros128)
    degb = jnp.broadcast_to((deg0[:N, 0] + deg1[:N, 0])[:, None], (N, D))

    xs, hs, disb = _stage_b(x, H, degb)

    sxr, shr = _prop2_kernel(xs, hs, src, dst, zeros128)
    sxr, shr = sxr[:N], shr[:N]

    wzr = jnp.concatenate([
        jnp.concatenate([W_xz[0], W_xr[0]], axis=1),
        jnp.concatenate([W_xz[1], W_xr[1]], axis=1),
        jnp.concatenate([W_hz[0], W_hr[0]], axis=1),
        jnp.concatenate([W_hz[1], W_hr[1]], axis=1),
    ], axis=0)
    bzr = jnp.concatenate([b_xz + b_hz, b_xr + b_hr])[None, :]
    wxh2 = jnp.concatenate([W_xh[0], W_xh[1]], axis=0)
    bh = (b_xh + b_hh)[None, :]
    z, p, gs = _stage_d(x, H, sxr, shr, disb, wzr, bzr, wxh2, W_hh[0], bh)

    sg0, sg1 = _prop1_kernel(gs, src, dst, zeros128)
    sg0, sg1 = sg0[:N], sg1[:N]

    wsum = (W_post[0] + W_post[1])[None, :]
    hn, h, hw = _stage_f(z, p, sg0, sg1, disb, H, W_hh[1], wsum)

    part = _pred_kernel(h, hw, srcl, dstl)

    part2d = part.reshape(EL * 16 // D, D)
    smat = jnp.repeat(jnp.eye(8, dtype=f32), 16, axis=0)
    csum8 = jnp.full((1, 8), b_post[0] + b_post[1], f32)
    pred = _stage_h(part2d, smat, csum8).reshape(EL)

    return (pred, hn)


# speedup vs baseline: 12.4232x; 1.3751x over previous
import functools

import jax
import jax.numpy as jnp
from jax import lax
from jax.experimental import pallas as pl
from jax.experimental.pallas import tpu as pltpu
from jax.experimental.pallas import tpu_sc as plsc

N = 10000
E = 320000
EL = 100000
D = 128

NC = 2
NS = 16
CB = 128
NCHUNKS_E = E // CB
N_PAD = 10240
ROWS_PER_TILE = N_PAD // NS
NB = 2
NBG = 2

_MESH = plsc.VectorSubcoreMesh(
    core_axis_name="c", subcore_axis_name="s", num_cores=NC, num_subcores=NS)

f32 = jnp.float32
i32 = jnp.int32


@functools.partial(
    pl.kernel,
    out_type=(jax.ShapeDtypeStruct((N_PAD, D), f32),
              jax.ShapeDtypeStruct((N_PAD, D), f32)),
    mesh=_MESH,
    scratch_types=[
        pltpu.VMEM((NB, CB), i32),
        pltpu.VMEM((CB, D), f32),
        pltpu.VMEM_SHARED((N_PAD, D), f32),
    ] + [pltpu.SemaphoreType.DMA] * (2 * NB),
)
def _deg_kernel(src_hbm, ones_hbm, zeros_hbm, deg0_hbm, deg1_hbm,
                idx_v, ones_v, acc_sh, *sems):
    sem_i = sems[0:NB]
    sem_s = sems[NB:2 * NB]
    c = lax.axis_index("c")
    s = lax.axis_index("s")
    wid = c * NS + s
    stride = NC * NS

    pltpu.sync_copy(zeros_hbm.at[pl.ds(0, ROWS_PER_TILE)],
                    acc_sh.at[pl.ds(s * ROWS_PER_TILE, ROWS_PER_TILE)])
    pltpu.sync_copy(ones_hbm, ones_v)
    plsc.subcore_barrier()

    cnt = (NCHUNKS_E - wid + (stride - 1)) // stride
    nsup = cnt // NB

    def body(i, _):
        descs = []
        for k in range(NB):
            base = (wid + (i * NB + k) * stride) * CB
            descs.append(pltpu.async_copy(src_hbm.at[pl.ds(base, CB)],
                                          idx_v.at[k], sem_i[k]))
        sd = []
        for k in range(NB):
            descs[k].wait()
            sd.append(pltpu.async_copy(ones_v, acc_sh.at[idx_v.at[k]],
                                       sem_s[k], add=True))
        for k in range(NB):
            sd[k].wait()
        return 0
    lax.fori_loop(0, nsup, body, 0)

    def tail(j, _):
        base = (wid + (nsup * NB + j) * stride) * CB
        pltpu.sync_copy(src_hbm.at[pl.ds(base, CB)], idx_v.at[0])
        pltpu.sync_copy(ones_v, acc_sh.at[idx_v.at[0]], add=True)
        return 0
    lax.fori_loop(0, cnt - nsup * NB, tail, 0)
    plsc.subcore_barrier()

    @pl.when(c == 0)
    def _():
        pltpu.sync_copy(acc_sh.at[pl.ds(s * ROWS_PER_TILE, ROWS_PER_TILE)],
                        deg0_hbm.at[pl.ds(s * ROWS_PER_TILE, ROWS_PER_TILE)])

    @pl.when(c == 1)
    def _():
        pltpu.sync_copy(acc_sh.at[pl.ds(s * ROWS_PER_TILE, ROWS_PER_TILE)],
                        deg1_hbm.at[pl.ds(s * ROWS_PER_TILE, ROWS_PER_TILE)])


@functools.partial(
    pl.kernel,
    out_type=(jax.ShapeDtypeStruct((N_PAD, D), f32),
              jax.ShapeDtypeStruct((N_PAD, D), f32)),
    mesh=_MESH,
    scratch_types=[
        pltpu.VMEM((NB, CB), i32),
        pltpu.VMEM((NB, CB), i32),
        pltpu.VMEM((NB, CB, D), f32),
        pltpu.VMEM_SHARED((N_PAD, D), f32),
    ] + [pltpu.SemaphoreType.DMA] * (3 * NB),
)
def _prop2_kernel(xs_hbm, hs_hbm, src_hbm, dst_hbm, zeros_hbm,
                  outx_hbm, outh_hbm, idxs_v, idxd_v, rows_v, acc_sh, *sems):
    sem_i = sems[0:NB]
    sem_g = sems[NB:2 * NB]
    sem_s = sems[2 * NB:3 * NB]
    c = lax.axis_index("c")
    s = lax.axis_index("s")

    pltpu.sync_copy(zeros_hbm,
                    acc_sh.at[pl.ds(s * ROWS_PER_TILE, ROWS_PER_TILE)])
    plsc.subcore_barrier()

    def run(table_hbm):
        cnt = (NCHUNKS_E - s + (NS - 1)) // NS
        nsup = cnt // NB

        def body(i, _):
            descs = []
            for k in range(NB):
                base = (s + (i * NB + k) * NS) * CB
                di = pltpu.async_copy(src_hbm.at[pl.ds(base, CB)],
                                      idxs_v.at[k], sem_i[k])
                dj = pltpu.async_copy(dst_hbm.at[pl.ds(base, CB)],
                                      idxd_v.at[k], sem_i[k])
                descs.append((di, dj))
            gd = []
            for k in range(NB):
                descs[k][0].wait()
                descs[k][1].wait()
                gd.append(pltpu.async_copy(table_hbm.at[idxs_v.at[k]],
                                           rows_v.at[k], sem_g[k]))
            sd = []
            for k in range(NB):
                gd[k].wait()
                sd.append(pltpu.async_copy(rows_v.at[k],
                                           acc_sh.at[idxd_v.at[k]],
                                           sem_s[k], add=True))
            for k in range(NB):
                sd[k].wait()
            return 0
        lax.fori_loop(0, nsup, body, 0)

        def tail(j, _):
            base = (s + (nsup * NB + j) * NS) * CB
            pltpu.sync_copy(src_hbm.at[pl.ds(base, CB)], idxs_v.at[0])
            pltpu.sync_copy(dst_hbm.at[pl.ds(base, CB)], idxd_v.at[0])
            pltpu.async_copy(table_hbm.at[idxs_v.at[0]],
                             rows_v.at[0], sem_g[0]).wait()
            pltpu.sync_copy(rows_v.at[0], acc_sh.at[idxd_v.at[0]], add=True)
            return 0
        lax.fori_loop(0, cnt - nsup * NB, tail, 0)

    @pl.when(c == 0)
    def _():
        run(xs_hbm)

    @pl.when(c == 1)
    def _():
        run(hs_hbm)

    plsc.subcore_barrier()

    @pl.when(c == 0)
    def _():
        pltpu.sync_copy(acc_sh.at[pl.ds(s * ROWS_PER_TILE, ROWS_PER_TILE)],
                        outx_hbm.at[pl.ds(s * ROWS_PER_TILE, ROWS_PER_TILE)])

    @pl.when(c == 1)
    def _():
        pltpu.sync_copy(acc_sh.at[pl.ds(s * ROWS_PER_TILE, ROWS_PER_TILE)],
                        outh_hbm.at[pl.ds(s * ROWS_PER_TILE, ROWS_PER_TILE)])


@functools.partial(
    pl.kernel,
    out_type=(jax.ShapeDtypeStruct((N_PAD, D), f32),
              jax.ShapeDtypeStruct((N_PAD, D), f32)),
    mesh=_MESH,
    scratch_types=[
        pltpu.VMEM((NB, CB), i32),
        pltpu.VMEM((NB, CB), i32),
        pltpu.VMEM((NB, CB, D), f32),
        pltpu.VMEM_SHARED((N_PAD, D), f32),
    ] + [pltpu.SemaphoreType.DMA] * (3 * NB),
)
def _prop1_kernel(gs_hbm, src_hbm, dst_hbm, zeros_hbm,
                  out0_hbm, out1_hbm, idxs_v, idxd_v, rows_v, acc_sh, *sems):
    sem_i = sems[0:NB]
    sem_g = sems[NB:2 * NB]
    sem_s = sems[2 * NB:3 * NB]
    c = lax.axis_index("c")
    s = lax.axis_index("s")
    wid = c * NS + s
    stride = NC * NS

    pltpu.sync_copy(zeros_hbm,
                    acc_sh.at[pl.ds(s * ROWS_PER_TILE, ROWS_PER_TILE)])
    plsc.subcore_barrier()

    cnt = (NCHUNKS_E - wid + (stride - 1)) // stride
    nsup = cnt // NB

    def body(i, _):
        descs = []
        for k in range(NB):
            base = (wid + (i * NB + k) * stride) * CB
            di = pltpu.async_copy(src_hbm.at[pl.ds(base, CB)],
                                  idxs_v.at[k], sem_i[k])
            dj = pltpu.async_copy(dst_hbm.at[pl.ds(base, CB)],
                                  idxd_v.at[k], sem_i[k])
            descs.append((di, dj))
        gd = []
        for k in range(NB):
            descs[k][0].wait()
            descs[k][1].wait()
            gd.append(pltpu.async_copy(gs_hbm.at[idxs_v.at[k]],
                                       rows_v.at[k], sem_g[k]))
        sd = []
        for k in range(NB):
            gd[k].wait()
            sd.append(pltpu.async_copy(rows_v.at[k],
                                       acc_sh.at[idxd_v.at[k]],
                                       sem_s[k], add=True))
        for k in range(NB):
            sd[k].wait()
        return 0
    lax.fori_loop(0, nsup, body, 0)

    def tail(j, _):
        base = (wid + (nsup * NB + j) * stride) * CB
        pltpu.sync_copy(src_hbm.at[pl.ds(base, CB)], idxs_v.at[0])
        pltpu.sync_copy(dst_hbm.at[pl.ds(base, CB)], idxd_v.at[0])
        pltpu.async_copy(gs_hbm.at[idxs_v.at[0]],
                         rows_v.at[0], sem_g[0]).wait()
        pltpu.sync_copy(rows_v.at[0], acc_sh.at[idxd_v.at[0]], add=True)
        return 0
    lax.fori_loop(0, cnt - nsup * NB, tail, 0)
    plsc.subcore_barrier()

    @pl.when(c == 0)
    def _():
        pltpu.sync_copy(acc_sh.at[pl.ds(s * ROWS_PER_TILE, ROWS_PER_TILE)],
                        out0_hbm.at[pl.ds(s * ROWS_PER_TILE, ROWS_PER_TILE)])

    @pl.when(c == 1)
    def _():
        pltpu.sync_copy(acc_sh.at[pl.ds(s * ROWS_PER_TILE, ROWS_PER_TILE)],
                        out1_hbm.at[pl.ds(s * ROWS_PER_TILE, ROWS_PER_TILE)])


NCHUNKS_L = (EL + CB - 1) // CB
LAST_START = EL - CB


@functools.partial(
    pl.kernel,
    out_type=jax.ShapeDtypeStruct((EL, 16), f32),
    mesh=_MESH,
    scratch_types=[
        pltpu.VMEM((NBG, CB), i32),
        pltpu.VMEM((NBG, CB), i32),
        pltpu.VMEM((NBG, CB, D), f32),
        pltpu.VMEM((NBG, CB, D), f32),
        pltpu.VMEM((NBG, CB, 16), f32),
    ] + [pltpu.SemaphoreType.DMA] * (3 * NBG),
)
def _pred_kernel(h_hbm, hw_hbm, srcl_hbm, dstl_hbm, part_hbm,
                 idxa_v, idxb_v, ra_v, rb_v, part_v, *sems):
    sem_i = sems[0:NBG]
    sem_g = sems[NBG:2 * NBG]
    sem_w = sems[2 * NBG:3 * NBG]
    c = lax.axis_index("c")
    s = lax.axis_index("s")
    wid = c * NS + s
    stride = NC * NS

    cnt = (NCHUNKS_L - wid + (stride - 1)) // stride
    nsup = cnt // NBG

    def compute(k, start):
        def ebody(e, _):
            acc = ra_v[k, e, pl.ds(0, 16)] * rb_v[k, e, pl.ds(0, 16)]
            for q in range(1, D // 16):
                acc = acc + (ra_v[k, e, pl.ds(q * 16, 16)]
                             * rb_v[k, e, pl.ds(q * 16, 16)])
            part_v[k, e, :] = acc
            return 0
        lax.fori_loop(0, CB, ebody, 0)
        return pltpu.async_copy(part_v.at[k], part_hbm.at[pl.ds(start, CB)],
                                sem_w[k])

    def body(i, _):
        starts = []
        descs = []
        for k in range(NBG):
            chunk = wid + (i * NBG + k) * stride
            start = pl.multiple_of(jnp.minimum(chunk * CB, LAST_START), 32)
            starts.append(start)
            da = pltpu.async_copy(srcl_hbm.at[pl.ds(start, CB)],
                                  idxa_v.at[k], sem_i[k])
            db = pltpu.async_copy(dstl_hbm.at[pl.ds(start, CB)],
                                  idxb_v.at[k], sem_i[k])
            descs.append((da, db))
        gd = []
        for k in range(NBG):
            descs[k][0].wait()
            descs[k][1].wait()
            ga = pltpu.async_copy(h_hbm.at[idxa_v.at[k]], ra_v.at[k],
                                  sem_g[k])
            gb = pltpu.async_copy(hw_hbm.at[idxb_v.at[k]], rb_v.at[k],
                                  sem_g[k])
            gd.append((ga, gb))
        wd = []
        for k in range(NBG):
            gd[k][0].wait()
            gd[k][1].wait()
            wd.append(compute(k, starts[k]))
        for k in range(NBG):
            wd[k].wait()
        return 0
    lax.fori_loop(0, nsup, body, 0)

    def tail(j, _):
        chunk = wid + (nsup * NBG + j) * stride
        start = pl.multiple_of(jnp.minimum(chunk * CB, LAST_START), 32)
        pltpu.sync_copy(srcl_hbm.at[pl.ds(start, CB)], idxa_v.at[0])
        pltpu.sync_copy(dstl_hbm.at[pl.ds(start, CB)], idxb_v.at[0])
        pltpu.async_copy(h_hbm.at[idxa_v.at[0]], ra_v.at[0], sem_g[0]).wait()
        pltpu.async_copy(hw_hbm.at[idxb_v.at[0]], rb_v.at[0], sem_g[0]).wait()
        compute(0, start).wait()
        return 0
    lax.fori_loop(0, cnt - nsup * NBG, tail, 0)


def _reduce_body(p_ref, s_ref, c_ref, out_ref):
    out_ref[...] = (jnp.dot(p_ref[...], s_ref[...], preferred_element_type=f32)
                    + c_ref[...])


def _stage_h(part2d, smat, csum8):
    nrows = EL * 16 // D
    return pl.pallas_call(
        _reduce_body,
        out_shape=jax.ShapeDtypeStruct((nrows, 8), f32),
    )(part2d, smat, csum8)


RB = 1000
GRID = N // RB


def _row_spec():
    return pl.BlockSpec((RB, D), lambda i: (i, 0))


def _full_spec(shape):
    return pl.BlockSpec(shape, lambda i: tuple(0 for _ in shape))


def _scale_body(x_ref, h_ref, degb_ref, xs_ref, hs_ref, disb_ref):
    deg = degb_ref[...]
    dis = jnp.where(deg > 0, lax.rsqrt(jnp.maximum(deg, 1e-12)), 0.0)
    xs_ref[...] = -(x_ref[...] * dis)
    hs_ref[...] = -(h_ref[...] * dis)
    disb_ref[...] = dis


def _stage_b(x, H, degb):
    return pl.pallas_call(
        _scale_body,
        grid=(GRID,),
        in_specs=[_row_spec()] * 3,
        out_specs=[_row_spec()] * 3,
        out_shape=[jax.ShapeDtypeStruct((N, D), f32)] * 3,
    )(x, H, degb)


def _gates_body(x_ref, h_ref, sxr_ref, shr_ref, disb_ref,
                wzr_ref, bzr_ref, wxh2_ref, whh0_ref, bh_ref,
                z_ref, p_ref, gs_ref):
    dis = disb_ref[...]
    xv = x_ref[...]
    hv = h_ref[...]
    sx = sxr_ref[...] * dis
    sh = shr_ref[...] * dis
    cat = jnp.concatenate([xv, sx, hv, sh], axis=1)
    zr = jnp.dot(cat, wzr_ref[...], preferred_element_type=f32) + bzr_ref[...]
    z = jax.nn.sigmoid(zr[:, :D])
    r = jax.nn.sigmoid(zr[:, D:])
    g = hv * r
    p = (jnp.dot(jnp.concatenate([xv, sx], axis=1), wxh2_ref[...],
                 preferred_element_type=f32)
         + jnp.dot(g, whh0_ref[...], preferred_element_type=f32)
         + bh_ref[...])
    z_ref[...] = z
    p_ref[...] = p
    gs_ref[...] = -(g * dis)


def _stage_d(x, H, sxr, shr, disb, wzr, bzr, wxh2, whh0, bh):
    return pl.pallas_call(
        _gates_body,
        grid=(GRID,),
        in_specs=[_row_spec()] * 5 + [
            _full_spec((4 * D, 2 * D)), _full_spec((1, 2 * D)),
            _full_spec((2 * D, D)), _full_spec((D, D)), _full_spec((1, D)),
        ],
        out_specs=[_row_spec()] * 3,
        out_shape=[jax.ShapeDtypeStruct((N, D), f32)] * 3,
    )(x, H, sxr, shr, disb, wzr, bzr, wxh2, whh0, bh)


def _update_body(z_ref, p_ref, sg0_ref, sg1_ref, disb_ref, h_ref,
                 whh1_ref, wsum_ref, hn_ref, hr_ref, hwr_ref):
    sg = (sg0_ref[...] + sg1_ref[...]) * disb_ref[...]
    ht = jnp.tanh(p_ref[...] + jnp.dot(sg, whh1_ref[...],
                                       preferred_element_type=f32))
    z = z_ref[...]
    hn = z * h_ref[...] + (1.0 - z) * ht
    hrelu = jnp.maximum(hn, 0.0)
    hn_ref[...] = hn
    hr_ref[...] = hrelu
    hwr_ref[...] = hrelu * wsum_ref[...]


def _stage_f(z, p, sg0, sg1, disb, H, whh1, wsum):
    return pl.pallas_call(
        _update_body,
        grid=(GRID,),
        in_specs=[_row_spec()] * 6 + [_full_spec((D, D)), _full_spec((1, D))],
        out_specs=[_row_spec()] * 3,
        out_shape=[jax.ShapeDtypeStruct((N, D), f32)] * 3,
    )(z, p, sg0, sg1, disb, H, whh1, wsum)


def kernel(x, edge_index, edge_label_index, H,
           W_xz, b_xz, W_hz, b_hz, W_xr, b_xr, W_hr, b_hr,
           W_xh, b_xh, W_hh, b_hh, W_post, b_post):
    src = edge_index[0]
    dst = edge_index[1]
    srcl = edge_label_index[0]
    dstl = edge_label_index[1]

    zeros128 = jnp.zeros((ROWS_PER_TILE, D), f32)

    ones128 = jnp.ones((CB, D), f32)
    deg0, deg1 = _deg_kernel(src, ones128, zeros128)
    degb = jnp.broadcast_to((deg0[:N, 0] + deg1[:N, 0])[:, None], (N, D))

    xs, hs, disb = _stage_b(x, H, degb)

    sxr, shr = _prop2_kernel(xs, hs, src, dst, zeros128)
    sxr, shr = sxr[:N], shr[:N]

    wzr = jnp.concatenate([
        jnp.concatenate([W_xz[0], W_xr[0]], axis=1),
        jnp.concatenate([W_xz[1], W_xr[1]], axis=1),
        jnp.concatenate([W_hz[0], W_hr[0]], axis=1),
        jnp.concatenate([W_hz[1], W_hr[1]], axis=1),
    ], axis=0)
    bzr = jnp.concatenate([b_xz + b_hz, b_xr + b_hr])[None, :]
    wxh2 = jnp.concatenate([W_xh[0], W_xh[1]], axis=0)
    bh = (b_xh + b_hh)[None, :]
    z, p, gs = _stage_d(x, H, sxr, shr, disb, wzr, bzr, wxh2, W_hh[0], bh)

    sg0, sg1 = _prop1_kernel(gs, src, dst, zeros128)
    sg0, sg1 = sg0[:N], sg1[:N]

    wsum = (W_post[0] + W_post[1])[None, :]
    hn, h, hw = _stage_f(z, p, sg0, sg1, disb, H, W_hh[1], wsum)

    part = _pred_kernel(h, hw, srcl, dstl)

    part2d = part.reshape(EL * 16 // D, D)
    smat = jnp.repeat(jnp.eye(8, dtype=f32), 16, axis=0)
    csum8 = jnp.full((1, 8), b_post[0] + b_post[1], f32)
    pred = _stage_h(part2d, smat, csum8).reshape(EL)

    return (pred, hn)


# speedup vs baseline: 13.0532x; 1.0507x over previous
import functools

import jax
import jax.numpy as jnp
from jax import lax
from jax.experimental import pallas as pl
from jax.experimental.pallas import tpu as pltpu
from jax.experimental.pallas import tpu_sc as plsc

N = 10000
E = 320000
EL = 100000
D = 128

NC = 2
NS = 16
CB = 128
NCHUNKS_E = E // CB
N_PAD = 10240
ROWS_PER_TILE = N_PAD // NS
NB = 2
CBP = 64
NBP = 5
NCHUNKS_P = E // CBP
NBG = 2

_MESH = plsc.VectorSubcoreMesh(
    core_axis_name="c", subcore_axis_name="s", num_cores=NC, num_subcores=NS)

f32 = jnp.float32
i32 = jnp.int32


@functools.partial(
    pl.kernel,
    out_type=(jax.ShapeDtypeStruct((N_PAD, D), f32),
              jax.ShapeDtypeStruct((N_PAD, D), f32)),
    mesh=_MESH,
    scratch_types=[
        pltpu.VMEM((NB, CB), i32),
        pltpu.VMEM((CB, D), f32),
        pltpu.VMEM_SHARED((N_PAD, D), f32),
    ] + [pltpu.SemaphoreType.DMA] * (2 * NB),
)
def _deg_kernel(src_hbm, ones_hbm, zeros_hbm, deg0_hbm, deg1_hbm,
                idx_v, ones_v, acc_sh, *sems):
    sem_i = sems[0:NB]
    sem_s = sems[NB:2 * NB]
    c = lax.axis_index("c")
    s = lax.axis_index("s")
    wid = c * NS + s
    stride = NC * NS

    pltpu.sync_copy(zeros_hbm.at[pl.ds(0, ROWS_PER_TILE)],
                    acc_sh.at[pl.ds(s * ROWS_PER_TILE, ROWS_PER_TILE)])
    pltpu.sync_copy(ones_hbm, ones_v)
    plsc.subcore_barrier()

    cnt = (NCHUNKS_E - wid + (stride - 1)) // stride
    nsup = cnt // NB

    def body(i, _):
        descs = []
        for k in range(NB):
            base = (wid + (i * NB + k) * stride) * CB
            descs.append(pltpu.async_copy(src_hbm.at[pl.ds(base, CB)],
                                          idx_v.at[k], sem_i[k]))
        sd = []
        for k in range(NB):
            descs[k].wait()
            sd.append(pltpu.async_copy(ones_v, acc_sh.at[idx_v.at[k]],
                                       sem_s[k], add=True))
        for k in range(NB):
            sd[k].wait()
        return 0
    lax.fori_loop(0, nsup, body, 0)

    def tail(j, _):
        base = (wid + (nsup * NB + j) * stride) * CB
        pltpu.sync_copy(src_hbm.at[pl.ds(base, CB)], idx_v.at[0])
        pltpu.sync_copy(ones_v, acc_sh.at[idx_v.at[0]], add=True)
        return 0
    lax.fori_loop(0, cnt - nsup * NB, tail, 0)
    plsc.subcore_barrier()

    @pl.when(c == 0)
    def _():
        pltpu.sync_copy(acc_sh.at[pl.ds(s * ROWS_PER_TILE, ROWS_PER_TILE)],
                        deg0_hbm.at[pl.ds(s * ROWS_PER_TILE, ROWS_PER_TILE)])

    @pl.when(c == 1)
    def _():
        pltpu.sync_copy(acc_sh.at[pl.ds(s * ROWS_PER_TILE, ROWS_PER_TILE)],
                        deg1_hbm.at[pl.ds(s * ROWS_PER_TILE, ROWS_PER_TILE)])


@functools.partial(
    pl.kernel,
    out_type=(jax.ShapeDtypeStruct((N_PAD, D), f32),
              jax.ShapeDtypeStruct((N_PAD, D), f32)),
    mesh=_MESH,
    scratch_types=[
        pltpu.VMEM((NBP, CBP), i32),
        pltpu.VMEM((NBP, CBP), i32),
        pltpu.VMEM((NBP, CBP, D), f32),
        pltpu.VMEM_SHARED((N_PAD, D), f32),
    ] + [pltpu.SemaphoreType.DMA] * (3 * NBP),
)
def _prop2_kernel(xs_hbm, hs_hbm, src_hbm, dst_hbm, zeros_hbm,
                  outx_hbm, outh_hbm, idxs_v, idxd_v, rows_v, acc_sh, *sems):
    sem_i = sems[0:NBP]
    sem_g = sems[NBP:2 * NBP]
    sem_s = sems[2 * NBP:3 * NBP]
    c = lax.axis_index("c")
    s = lax.axis_index("s")

    pltpu.sync_copy(zeros_hbm,
                    acc_sh.at[pl.ds(s * ROWS_PER_TILE, ROWS_PER_TILE)])
    plsc.subcore_barrier()

    def run(table_hbm):
        cnt = (NCHUNKS_P - s + (NS - 1)) // NS
        nsup = cnt // NBP

        def body(i, _):
            descs = []
            for k in range(NBP):
                base = (s + (i * NBP + k) * NS) * CBP
                di = pltpu.async_copy(src_hbm.at[pl.ds(base, CBP)],
                                      idxs_v.at[k], sem_i[k])
                dj = pltpu.async_copy(dst_hbm.at[pl.ds(base, CBP)],
                                      idxd_v.at[k], sem_i[k])
                descs.append((di, dj))
            gd = []
            for k in range(NBP):
                descs[k][0].wait()
                descs[k][1].wait()
                gd.append(pltpu.async_copy(table_hbm.at[idxs_v.at[k]],
                                           rows_v.at[k], sem_g[k]))
            sd = []
            for k in range(NBP):
                gd[k].wait()
                sd.append(pltpu.async_copy(rows_v.at[k],
                                           acc_sh.at[idxd_v.at[k]],
                                           sem_s[k], add=True))
            for k in range(NBP):
                sd[k].wait()
            return 0
        lax.fori_loop(0, nsup, body, 0)

        def tail(j, _):
            base = (s + (nsup * NBP + j) * NS) * CBP
            pltpu.sync_copy(src_hbm.at[pl.ds(base, CBP)], idxs_v.at[0])
            pltpu.sync_copy(dst_hbm.at[pl.ds(base, CBP)], idxd_v.at[0])
            pltpu.async_copy(table_hbm.at[idxs_v.at[0]],
                             rows_v.at[0], sem_g[0]).wait()
            pltpu.sync_copy(rows_v.at[0], acc_sh.at[idxd_v.at[0]], add=True)
            return 0
        lax.fori_loop(0, cnt - nsup * NBP, tail, 0)

    @pl.when(c == 0)
    def _():
        run(xs_hbm)

    @pl.when(c == 1)
    def _():
        run(hs_hbm)

    plsc.subcore_barrier()

    @pl.when(c == 0)
    def _():
        pltpu.sync_copy(acc_sh.at[pl.ds(s * ROWS_PER_TILE, ROWS_PER_TILE)],
                        outx_hbm.at[pl.ds(s * ROWS_PER_TILE, ROWS_PER_TILE)])

    @pl.when(c == 1)
    def _():
        pltpu.sync_copy(acc_sh.at[pl.ds(s * ROWS_PER_TILE, ROWS_PER_TILE)],
                        outh_hbm.at[pl.ds(s * ROWS_PER_TILE, ROWS_PER_TILE)])


@functools.partial(
    pl.kernel,
    out_type=(jax.ShapeDtypeStruct((N_PAD, D), f32),
              jax.ShapeDtypeStruct((N_PAD, D), f32)),
    mesh=_MESH,
    scratch_types=[
        pltpu.VMEM((NBP, CBP), i32),
        pltpu.VMEM((NBP, CBP), i32),
        pltpu.VMEM((NBP, CBP, D), f32),
        pltpu.VMEM_SHARED((N_PAD, D), f32),
    ] + [pltpu.SemaphoreType.DMA] * (3 * NBP),
)
def _prop1_kernel(gs_hbm, src_hbm, dst_hbm, zeros_hbm,
                  out0_hbm, out1_hbm, idxs_v, idxd_v, rows_v, acc_sh, *sems):
    sem_i = sems[0:NBP]
    sem_g = sems[NBP:2 * NBP]
    sem_s = sems[2 * NBP:3 * NBP]
    c = lax.axis_index("c")
    s = lax.axis_index("s")
    wid = c * NS + s
    stride = NC * NS

    pltpu.sync_copy(zeros_hbm,
                    acc_sh.at[pl.ds(s * ROWS_PER_TILE, ROWS_PER_TILE)])
    plsc.subcore_barrier()

    cnt = (NCHUNKS_P - wid + (stride - 1)) // stride
    nsup = cnt // NBP

    def body(i, _):
        descs = []
        for k in range(NBP):
            base = (wid + (i * NBP + k) * stride) * CBP
            di = pltpu.async_copy(src_hbm.at[pl.ds(base, CBP)],
                                  idxs_v.at[k], sem_i[k])
            dj = pltpu.async_copy(dst_hbm.at[pl.ds(base, CBP)],
                                  idxd_v.at[k], sem_i[k])
            descs.append((di, dj))
        gd = []
        for k in range(NBP):
            descs[k][0].wait()
            descs[k][1].wait()
            gd.append(pltpu.async_copy(gs_hbm.at[idxs_v.at[k]],
                                       rows_v.at[k], sem_g[k]))
        sd = []
        for k in range(NBP):
            gd[k].wait()
            sd.append(pltpu.async_copy(rows_v.at[k],
                                       acc_sh.at[idxd_v.at[k]],
                                       sem_s[k], add=True))
        for k in range(NBP):
            sd[k].wait()
        return 0
    lax.fori_loop(0, nsup, body, 0)

    def tail(j, _):
        base = (wid + (nsup * NBP + j) * stride) * CBP
        pltpu.sync_copy(src_hbm.at[pl.ds(base, CBP)], idxs_v.at[0])
        pltpu.sync_copy(dst_hbm.at[pl.ds(base, CBP)], idxd_v.at[0])
        pltpu.async_copy(gs_hbm.at[idxs_v.at[0]],
                         rows_v.at[0], sem_g[0]).wait()
        pltpu.sync_copy(rows_v.at[0], acc_sh.at[idxd_v.at[0]], add=True)
        return 0
    lax.fori_loop(0, cnt - nsup * NBP, tail, 0)
    plsc.subcore_barrier()

    @pl.when(c == 0)
    def _():
        pltpu.sync_copy(acc_sh.at[pl.ds(s * ROWS_PER_TILE, ROWS_PER_TILE)],
                        out0_hbm.at[pl.ds(s * ROWS_PER_TILE, ROWS_PER_TILE)])

    @pl.when(c == 1)
    def _():
        pltpu.sync_copy(acc_sh.at[pl.ds(s * ROWS_PER_TILE, ROWS_PER_TILE)],
                        out1_hbm.at[pl.ds(s * ROWS_PER_TILE, ROWS_PER_TILE)])


NCHUNKS_L = (EL + CB - 1) // CB
LAST_START = EL - CB


@functools.partial(
    pl.kernel,
    out_type=jax.ShapeDtypeStruct((EL, 16), f32),
    mesh=_MESH,
    scratch_types=[
        pltpu.VMEM((NBG, CB), i32),
        pltpu.VMEM((NBG, CB), i32),
        pltpu.VMEM((NBG, CB, D), f32),
        pltpu.VMEM((NBG, CB, D), f32),
        pltpu.VMEM((NBG, CB, 16), f32),
    ] + [pltpu.SemaphoreType.DMA] * (3 * NBG),
)
def _pred_kernel(h_hbm, hw_hbm, srcl_hbm, dstl_hbm, part_hbm,
                 idxa_v, idxb_v, ra_v, rb_v, part_v, *sems):
    sem_i = sems[0:NBG]
    sem_g = sems[NBG:2 * NBG]
    sem_w = sems[2 * NBG:3 * NBG]
    c = lax.axis_index("c")
    s = lax.axis_index("s")
    wid = c * NS + s
    stride = NC * NS

    cnt = (NCHUNKS_L - wid + (stride - 1)) // stride
    nsup = cnt // NBG

    def compute(k, start):
        def ebody(e, _):
            acc = ra_v[k, e, pl.ds(0, 16)] * rb_v[k, e, pl.ds(0, 16)]
            for q in range(1, D // 16):
                acc = acc + (ra_v[k, e, pl.ds(q * 16, 16)]
                             * rb_v[k, e, pl.ds(q * 16, 16)])
            part_v[k, e, :] = acc
            return 0
        lax.fori_loop(0, CB, ebody, 0)
        return pltpu.async_copy(part_v.at[k], part_hbm.at[pl.ds(start, CB)],
                                sem_w[k])

    def body(i, _):
        starts = []
        descs = []
        for k in range(NBG):
            chunk = wid + (i * NBG + k) * stride
            start = pl.multiple_of(jnp.minimum(chunk * CB, LAST_START), 32)
            starts.append(start)
            da = pltpu.async_copy(srcl_hbm.at[pl.ds(start, CB)],
                                  idxa_v.at[k], sem_i[k])
            db = pltpu.async_copy(dstl_hbm.at[pl.ds(start, CB)],
                                  idxb_v.at[k], sem_i[k])
            descs.append((da, db))
        gd = []
        for k in range(NBG):
            descs[k][0].wait()
            descs[k][1].wait()
            ga = pltpu.async_copy(h_hbm.at[idxa_v.at[k]], ra_v.at[k],
                                  sem_g[k])
            gb = pltpu.async_copy(hw_hbm.at[idxb_v.at[k]], rb_v.at[k],
                                  sem_g[k])
            gd.append((ga, gb))
        wd = []
        for k in range(NBG):
            gd[k][0].wait()
            gd[k][1].wait()
            wd.append(compute(k, starts[k]))
        for k in range(NBG):
            wd[k].wait()
        return 0
    lax.fori_loop(0, nsup, body, 0)

    def tail(j, _):
        chunk = wid + (nsup * NBG + j) * stride
        start = pl.multiple_of(jnp.minimum(chunk * CB, LAST_START), 32)
        pltpu.sync_copy(srcl_hbm.at[pl.ds(start, CB)], idxa_v.at[0])
        pltpu.sync_copy(dstl_hbm.at[pl.ds(start, CB)], idxb_v.at[0])
        pltpu.async_copy(h_hbm.at[idxa_v.at[0]], ra_v.at[0], sem_g[0]).wait()
        pltpu.async_copy(hw_hbm.at[idxb_v.at[0]], rb_v.at[0], sem_g[0]).wait()
        compute(0, start).wait()
        return 0
    lax.fori_loop(0, cnt - nsup * NBG, tail, 0)


def _reduce_body(p_ref, s_ref, c_ref, out_ref):
    out_ref[...] = (jnp.dot(p_ref[...], s_ref[...], preferred_element_type=f32)
                    + c_ref[...])


def _stage_h(part2d, smat, csum8):
    nrows = EL * 16 // D
    return pl.pallas_call(
        _reduce_body,
        out_shape=jax.ShapeDtypeStruct((nrows, 8), f32),
    )(part2d, smat, csum8)


RB = 1000
GRID = N // RB


def _row_spec():
    return pl.BlockSpec((RB, D), lambda i: (i, 0))


def _full_spec(shape):
    return pl.BlockSpec(shape, lambda i: tuple(0 for _ in shape))


def _scale_body(x_ref, h_ref, degb_ref, xs_ref, hs_ref, disb_ref):
    deg = degb_ref[...]
    dis = jnp.where(deg > 0, lax.rsqrt(jnp.maximum(deg, 1e-12)), 0.0)
    xs_ref[...] = -(x_ref[...] * dis)
    hs_ref[...] = -(h_ref[...] * dis)
    disb_ref[...] = dis


def _stage_b(x, H, degb):
    return pl.pallas_call(
        _scale_body,
        grid=(GRID,),
        in_specs=[_row_spec()] * 3,
        out_specs=[_row_spec()] * 3,
        out_shape=[jax.ShapeDtypeStruct((N, D), f32)] * 3,
    )(x, H, degb)


def _gates_body(x_ref, h_ref, sxr_ref, shr_ref, disb_ref,
                wzr_ref, bzr_ref, wxh2_ref, whh0_ref, bh_ref,
                z_ref, p_ref, gs_ref):
    dis = disb_ref[...]
    xv = x_ref[...]
    hv = h_ref[...]
    sx = sxr_ref[...] * dis
    sh = shr_ref[...] * dis
    cat = jnp.concatenate([xv, sx, hv, sh], axis=1)
    zr = jnp.dot(cat, wzr_ref[...], preferred_element_type=f32) + bzr_ref[...]
    z = jax.nn.sigmoid(zr[:, :D])
    r = jax.nn.sigmoid(zr[:, D:])
    g = hv * r
    p = (jnp.dot(jnp.concatenate([xv, sx], axis=1), wxh2_ref[...],
                 preferred_element_type=f32)
         + jnp.dot(g, whh0_ref[...], preferred_element_type=f32)
         + bh_ref[...])
    z_ref[...] = z
    p_ref[...] = p
    gs_ref[...] = -(g * dis)


def _stage_d(x, H, sxr, shr, disb, wzr, bzr, wxh2, whh0, bh):
    return pl.pallas_call(
        _gates_body,
        grid=(GRID,),
        in_specs=[_row_spec()] * 5 + [
            _full_spec((4 * D, 2 * D)), _full_spec((1, 2 * D)),
            _full_spec((2 * D, D)), _full_spec((D, D)), _full_spec((1, D)),
        ],
        out_specs=[_row_spec()] * 3,
        out_shape=[jax.ShapeDtypeStruct((N, D), f32)] * 3,
    )(x, H, sxr, shr, disb, wzr, bzr, wxh2, whh0, bh)


def _update_body(z_ref, p_ref, sg0_ref, sg1_ref, disb_ref, h_ref,
                 whh1_ref, wsum_ref, hn_ref, hr_ref, hwr_ref):
    sg = (sg0_ref[...] + sg1_ref[...]) * disb_ref[...]
    ht = jnp.tanh(p_ref[...] + jnp.dot(sg, whh1_ref[...],
                                       preferred_element_type=f32))
    z = z_ref[...]
    hn = z * h_ref[...] + (1.0 - z) * ht
    hrelu = jnp.maximum(hn, 0.0)
    hn_ref[...] = hn
    hr_ref[...] = hrelu
    hwr_ref[...] = hrelu * wsum_ref[...]


def _stage_f(z, p, sg0, sg1, disb, H, whh1, wsum):
    return pl.pallas_call(
        _update_body,
        grid=(GRID,),
        in_specs=[_row_spec()] * 6 + [_full_spec((D, D)), _full_spec((1, D))],
        out_specs=[_row_spec()] * 3,
        out_shape=[jax.ShapeDtypeStruct((N, D), f32)] * 3,
    )(z, p, sg0, sg1, disb, H, whh1, wsum)


def kernel(x, edge_index, edge_label_index, H,
           W_xz, b_xz, W_hz, b_hz, W_xr, b_xr, W_hr, b_hr,
           W_xh, b_xh, W_hh, b_hh, W_post, b_post):
    src = edge_index[0]
    dst = edge_index[1]
    srcl = edge_label_index[0]
    dstl = edge_label_index[1]

    zeros128 = jnp.zeros((ROWS_PER_TILE, D), f32)

    ones128 = jnp.ones((CB, D), f32)
    deg0, deg1 = _deg_kernel(src, ones128, zeros128)
    degb = jnp.broadcast_to((deg0[:N, 0] + deg1[:N, 0])[:, None], (N, D))

    xs, hs, disb = _stage_b(x, H, degb)

    sxr, shr = _prop2_kernel(xs, hs, src, dst, zeros128)
    sxr, shr = sxr[:N], shr[:N]

    wzr = jnp.concatenate([
        jnp.concatenate([W_xz[0], W_xr[0]], axis=1),
        jnp.concatenate([W_xz[1], W_xr[1]], axis=1),
        jnp.concatenate([W_hz[0], W_hr[0]], axis=1),
        jnp.concatenate([W_hz[1], W_hr[1]], axis=1),
    ], axis=0)
    bzr = jnp.concatenate([b_xz + b_hz, b_xr + b_hr])[None, :]
    wxh2 = jnp.concatenate([W_xh[0], W_xh[1]], axis=0)
    bh = (b_xh + b_hh)[None, :]
    z, p, gs = _stage_d(x, H, sxr, shr, disb, wzr, bzr, wxh2, W_hh[0], bh)

    sg0, sg1 = _prop1_kernel(gs, src, dst, zeros128)
    sg0, sg1 = sg0[:N], sg1[:N]

    wsum = (W_post[0] + W_post[1])[None, :]
    hn, h, hw = _stage_f(z, p, sg0, sg1, disb, H, W_hh[1], wsum)

    part = _pred_kernel(h, hw, srcl, dstl)

    part2d = part.reshape(EL * 16 // D, D)
    smat = jnp.repeat(jnp.eye(8, dtype=f32), 16, axis=0)
    csum8 = jnp.full((1, 8), b_post[0] + b_post[1], f32)
    pred = _stage_h(part2d, smat, csum8).reshape(EL)

    return (pred, hn)


# speedup vs baseline: 13.1813x; 1.0098x over previous
import functools

import jax
import jax.numpy as jnp
from jax import lax
from jax.experimental import pallas as pl
from jax.experimental.pallas import tpu as pltpu
from jax.experimental.pallas import tpu_sc as plsc

N = 10000
E = 320000
EL = 100000
D = 128

NC = 2
NS = 16
CB = 128
NCHUNKS_E = E // CB
N_PAD = 10240
ROWS_PER_TILE = N_PAD // NS
NB = 6
CBP = 64
NBP = 5
NCHUNKS_P = E // CBP
NBG = 2

_MESH = plsc.VectorSubcoreMesh(
    core_axis_name="c", subcore_axis_name="s", num_cores=NC, num_subcores=NS)

f32 = jnp.float32
i32 = jnp.int32


@functools.partial(
    pl.kernel,
    out_type=(jax.ShapeDtypeStruct((N_PAD, D), f32),
              jax.ShapeDtypeStruct((N_PAD, D), f32)),
    mesh=_MESH,
    scratch_types=[
        pltpu.VMEM((NB, CB), i32),
        pltpu.VMEM((CB, D), f32),
        pltpu.VMEM_SHARED((N_PAD, D), f32),
    ] + [pltpu.SemaphoreType.DMA] * (2 * NB),
)
def _deg_kernel(src_hbm, ones_hbm, zeros_hbm, deg0_hbm, deg1_hbm,
                idx_v, ones_v, acc_sh, *sems):
    sem_i = sems[0:NB]
    sem_s = sems[NB:2 * NB]
    c = lax.axis_index("c")
    s = lax.axis_index("s")
    wid = c * NS + s
    stride = NC * NS

    pltpu.sync_copy(zeros_hbm.at[pl.ds(0, ROWS_PER_TILE)],
                    acc_sh.at[pl.ds(s * ROWS_PER_TILE, ROWS_PER_TILE)])
    pltpu.sync_copy(ones_hbm, ones_v)
    plsc.subcore_barrier()

    cnt = (NCHUNKS_E - wid + (stride - 1)) // stride
    nsup = cnt // NB

    def body(i, _):
        descs = []
        for k in range(NB):
            base = (wid + (i * NB + k) * stride) * CB
            descs.append(pltpu.async_copy(src_hbm.at[pl.ds(base, CB)],
                                          idx_v.at[k], sem_i[k]))
        sd = []
        for k in range(NB):
            descs[k].wait()
            sd.append(pltpu.async_copy(ones_v, acc_sh.at[idx_v.at[k]],
                                       sem_s[k], add=True))
        for k in range(NB):
            sd[k].wait()
        return 0
    lax.fori_loop(0, nsup, body, 0)

    def tail(j, _):
        base = (wid + (nsup * NB + j) * stride) * CB
        pltpu.sync_copy(src_hbm.at[pl.ds(base, CB)], idx_v.at[0])
        pltpu.sync_copy(ones_v, acc_sh.at[idx_v.at[0]], add=True)
        return 0
    lax.fori_loop(0, cnt - nsup * NB, tail, 0)
    plsc.subcore_barrier()

    @pl.when(c == 0)
    def _():
        pltpu.sync_copy(acc_sh.at[pl.ds(s * ROWS_PER_TILE, ROWS_PER_TILE)],
                        deg0_hbm.at[pl.ds(s * ROWS_PER_TILE, ROWS_PER_TILE)])

    @pl.when(c == 1)
    def _():
        pltpu.sync_copy(acc_sh.at[pl.ds(s * ROWS_PER_TILE, ROWS_PER_TILE)],
                        deg1_hbm.at[pl.ds(s * ROWS_PER_TILE, ROWS_PER_TILE)])


@functools.partial(
    pl.kernel,
    out_type=(jax.ShapeDtypeStruct((N_PAD, D), f32),
              jax.ShapeDtypeStruct((N_PAD, D), f32)),
    mesh=_MESH,
    scratch_types=[
        pltpu.VMEM((NBP, CBP), i32),
        pltpu.VMEM((NBP, CBP), i32),
        pltpu.VMEM((NBP, CBP, D), f32),
        pltpu.VMEM_SHARED((N_PAD, D), f32),
    ] + [pltpu.SemaphoreType.DMA] * (3 * NBP),
)
def _prop2_kernel(xs_hbm, hs_hbm, src_hbm, dst_hbm, zeros_hbm,
                  outx_hbm, outh_hbm, idxs_v, idxd_v, rows_v, acc_sh, *sems):
    sem_i = sems[0:NBP]
    sem_g = sems[NBP:2 * NBP]
    sem_s = sems[2 * NBP:3 * NBP]
    c = lax.axis_index("c")
    s = lax.axis_index("s")

    pltpu.sync_copy(zeros_hbm,
                    acc_sh.at[pl.ds(s * ROWS_PER_TILE, ROWS_PER_TILE)])
    plsc.subcore_barrier()

    def run(table_hbm):
        cnt = (NCHUNKS_P - s + (NS - 1)) // NS
        nsup = cnt // NBP

        def body(i, _):
            descs = []
            for k in range(NBP):
                base = (s + (i * NBP + k) * NS) * CBP
                di = pltpu.async_copy(src_hbm.at[pl.ds(base, CBP)],
                                      idxs_v.at[k], sem_i[k])
                dj = pltpu.async_copy(dst_hbm.at[pl.ds(base, CBP)],
                                      idxd_v.at[k], sem_i[k])
                descs.append((di, dj))
            gd = []
            for k in range(NBP):
                descs[k][0].wait()
                descs[k][1].wait()
                gd.append(pltpu.async_copy(table_hbm.at[idxs_v.at[k]],
                                           rows_v.at[k], sem_g[k]))
            sd = []
            for k in range(NBP):
                gd[k].wait()
                sd.append(pltpu.async_copy(rows_v.at[k],
                                           acc_sh.at[idxd_v.at[k]],
                                           sem_s[k], add=True))
            for k in range(NBP):
                sd[k].wait()
            return 0
        lax.fori_loop(0, nsup, body, 0)

        def tail(j, _):
            base = (s + (nsup * NBP + j) * NS) * CBP
            pltpu.sync_copy(src_hbm.at[pl.ds(base, CBP)], idxs_v.at[0])
            pltpu.sync_copy(dst_hbm.at[pl.ds(base, CBP)], idxd_v.at[0])
            pltpu.async_copy(table_hbm.at[idxs_v.at[0]],
                             rows_v.at[0], sem_g[0]).wait()
            pltpu.sync_copy(rows_v.at[0], acc_sh.at[idxd_v.at[0]], add=True)
            return 0
        lax.fori_loop(0, cnt - nsup * NBP, tail, 0)

    @pl.when(c == 0)
    def _():
        run(xs_hbm)

    @pl.when(c == 1)
    def _():
        run(hs_hbm)

    plsc.subcore_barrier()

    @pl.when(c == 0)
    def _():
        pltpu.sync_copy(acc_sh.at[pl.ds(s * ROWS_PER_TILE, ROWS_PER_TILE)],
                        outx_hbm.at[pl.ds(s * ROWS_PER_TILE, ROWS_PER_TILE)])

    @pl.when(c == 1)
    def _():
        pltpu.sync_copy(acc_sh.at[pl.ds(s * ROWS_PER_TILE, ROWS_PER_TILE)],
                        outh_hbm.at[pl.ds(s * ROWS_PER_TILE, ROWS_PER_TILE)])


@functools.partial(
    pl.kernel,
    out_type=(jax.ShapeDtypeStruct((N_PAD, D), f32),
              jax.ShapeDtypeStruct((N_PAD, D), f32)),
    mesh=_MESH,
    scratch_types=[
        pltpu.VMEM((NBP, CBP), i32),
        pltpu.VMEM((NBP, CBP), i32),
        pltpu.VMEM((NBP, CBP, D), f32),
        pltpu.VMEM_SHARED((N_PAD, D), f32),
    ] + [pltpu.SemaphoreType.DMA] * (3 * NBP),
)
def _prop1_kernel(gs_hbm, src_hbm, dst_hbm, zeros_hbm,
                  out0_hbm, out1_hbm, idxs_v, idxd_v, rows_v, acc_sh, *sems):
    sem_i = sems[0:NBP]
    sem_g = sems[NBP:2 * NBP]
    sem_s = sems[2 * NBP:3 * NBP]
    c = lax.axis_index("c")
    s = lax.axis_index("s")
    wid = c * NS + s
    stride = NC * NS

    pltpu.sync_copy(zeros_hbm,
                    acc_sh.at[pl.ds(s * ROWS_PER_TILE, ROWS_PER_TILE)])
    plsc.subcore_barrier()

    cnt = (NCHUNKS_P - wid + (stride - 1)) // stride
    nsup = cnt // NBP

    def body(i, _):
        descs = []
        for k in range(NBP):
            base = (wid + (i * NBP + k) * stride) * CBP
            di = pltpu.async_copy(src_hbm.at[pl.ds(base, CBP)],
                                  idxs_v.at[k], sem_i[k])
            dj = pltpu.async_copy(dst_hbm.at[pl.ds(base, CBP)],
                                  idxd_v.at[k], sem_i[k])
            descs.append((di, dj))
        gd = []
        for k in range(NBP):
            descs[k][0].wait()
            descs[k][1].wait()
            gd.append(pltpu.async_copy(gs_hbm.at[idxs_v.at[k]],
                                       rows_v.at[k], sem_g[k]))
        sd = []
        for k in range(NBP):
            gd[k].wait()
            sd.append(pltpu.async_copy(rows_v.at[k],
                                       acc_sh.at[idxd_v.at[k]],
                                       sem_s[k], add=True))
        for k in range(NBP):
            sd[k].wait()
        return 0
    lax.fori_loop(0, nsup, body, 0)

    def tail(j, _):
        base = (wid + (nsup * NBP + j) * stride) * CBP
        pltpu.sync_copy(src_hbm.at[pl.ds(base, CBP)], idxs_v.at[0])
        pltpu.sync_copy(dst_hbm.at[pl.ds(base, CBP)], idxd_v.at[0])
        pltpu.async_copy(gs_hbm.at[idxs_v.at[0]],
                         rows_v.at[0], sem_g[0]).wait()
        pltpu.sync_copy(rows_v.at[0], acc_sh.at[idxd_v.at[0]], add=True)
        return 0
    lax.fori_loop(0, cnt - nsup * NBP, tail, 0)
    plsc.subcore_barrier()

    @pl.when(c == 0)
    def _():
        pltpu.sync_copy(acc_sh.at[pl.ds(s * ROWS_PER_TILE, ROWS_PER_TILE)],
                        out0_hbm.at[pl.ds(s * ROWS_PER_TILE, ROWS_PER_TILE)])

    @pl.when(c == 1)
    def _():
        pltpu.sync_copy(acc_sh.at[pl.ds(s * ROWS_PER_TILE, ROWS_PER_TILE)],
                        out1_hbm.at[pl.ds(s * ROWS_PER_TILE, ROWS_PER_TILE)])


NCHUNKS_L = (EL + CB - 1) // CB
LAST_START = EL - CB


@functools.partial(
    pl.kernel,
    out_type=jax.ShapeDtypeStruct((EL, 16), f32),
    mesh=_MESH,
    scratch_types=[
        pltpu.VMEM((NBG, CB), i32),
        pltpu.VMEM((NBG, CB), i32),
        pltpu.VMEM((NBG, CB, D), f32),
        pltpu.VMEM((NBG, CB, D), f32),
        pltpu.VMEM((NBG, CB, 16), f32),
    ] + [pltpu.SemaphoreType.DMA] * (3 * NBG),
)
def _pred_kernel(h_hbm, hw_hbm, srcl_hbm, dstl_hbm, part_hbm,
                 idxa_v, idxb_v, ra_v, rb_v, part_v, *sems):
    sem_i = sems[0:NBG]
    sem_g = sems[NBG:2 * NBG]
    sem_w = sems[2 * NBG:3 * NBG]
    c = lax.axis_index("c")
    s = lax.axis_index("s")
    wid = c * NS + s
    stride = NC * NS

    cnt = (NCHUNKS_L - wid + (stride - 1)) // stride
    nsup = cnt // NBG

    def compute(k, start):
        def ebody(e, _):
            acc = ra_v[k, e, pl.ds(0, 16)] * rb_v[k, e, pl.ds(0, 16)]
            for q in range(1, D // 16):
                acc = acc + (ra_v[k, e, pl.ds(q * 16, 16)]
                             * rb_v[k, e, pl.ds(q * 16, 16)])
            part_v[k, e, :] = acc
            return 0
        lax.fori_loop(0, CB, ebody, 0)
        return pltpu.async_copy(part_v.at[k], part_hbm.at[pl.ds(start, CB)],
                                sem_w[k])

    def body(i, _):
        starts = []
        descs = []
        for k in range(NBG):
            chunk = wid + (i * NBG + k) * stride
            start = pl.multiple_of(jnp.minimum(chunk * CB, LAST_START), 32)
            starts.append(start)
            da = pltpu.async_copy(srcl_hbm.at[pl.ds(start, CB)],
                                  idxa_v.at[k], sem_i[k])
            db = pltpu.async_copy(dstl_hbm.at[pl.ds(start, CB)],
                                  idxb_v.at[k], sem_i[k])
            descs.append((da, db))
        gd = []
        for k in range(NBG):
            descs[k][0].wait()
            descs[k][1].wait()
            ga = pltpu.async_copy(h_hbm.at[idxa_v.at[k]], ra_v.at[k],
                                  sem_g[k])
            gb = pltpu.async_copy(hw_hbm.at[idxb_v.at[k]], rb_v.at[k],
                                  sem_g[k])
            gd.append((ga, gb))
        wd = []
        for k in range(NBG):
            gd[k][0].wait()
            gd[k][1].wait()
            wd.append(compute(k, starts[k]))
        for k in range(NBG):
            wd[k].wait()
        return 0
    lax.fori_loop(0, nsup, body, 0)

    def tail(j, _):
        chunk = wid + (nsup * NBG + j) * stride
        start = pl.multiple_of(jnp.minimum(chunk * CB, LAST_START), 32)
        pltpu.sync_copy(srcl_hbm.at[pl.ds(start, CB)], idxa_v.at[0])
        pltpu.sync_copy(dstl_hbm.at[pl.ds(start, CB)], idxb_v.at[0])
        pltpu.async_copy(h_hbm.at[idxa_v.at[0]], ra_v.at[0], sem_g[0]).wait()
        pltpu.async_copy(hw_hbm.at[idxb_v.at[0]], rb_v.at[0], sem_g[0]).wait()
        compute(0, start).wait()
        return 0
    lax.fori_loop(0, cnt - nsup * NBG, tail, 0)


def _reduce_body(p_ref, s_ref, c_ref, out_ref):
    out_ref[...] = (jnp.dot(p_ref[...], s_ref[...], preferred_element_type=f32)
                    + c_ref[...])


def _stage_h(part2d, smat, csum8):
    nrows = EL * 16 // D
    return pl.pallas_call(
        _reduce_body,
        out_shape=jax.ShapeDtypeStruct((nrows, 8), f32),
    )(part2d, smat, csum8)


RB = 1000
GRID = N // RB


def _row_spec():
    return pl.BlockSpec((RB, D), lambda i: (i, 0))


def _full_spec(shape):
    return pl.BlockSpec(shape, lambda i: tuple(0 for _ in shape))


def _scale_body(x_ref, h_ref, degb_ref, xs_ref, hs_ref, disb_ref):
    deg = degb_ref[...]
    dis = jnp.where(deg > 0, lax.rsqrt(jnp.maximum(deg, 1e-12)), 0.0)
    xs_ref[...] = -(x_ref[...] * dis)
    hs_ref[...] = -(h_ref[...] * dis)
    disb_ref[...] = dis


def _stage_b(x, H, degb):
    return pl.pallas_call(
        _scale_body,
        grid=(GRID,),
        in_specs=[_row_spec()] * 3,
        out_specs=[_row_spec()] * 3,
        out_shape=[jax.ShapeDtypeStruct((N, D), f32)] * 3,
    )(x, H, degb)


def _gates_body(x_ref, h_ref, sxr_ref, shr_ref, disb_ref,
                wzr_ref, bzr_ref, wxh2_ref, whh0_ref, bh_ref,
                z_ref, p_ref, gs_ref):
    dis = disb_ref[...]
    xv = x_ref[...]
    hv = h_ref[...]
    sx = sxr_ref[...] * dis
    sh = shr_ref[...] * dis
    cat = jnp.concatenate([xv, sx, hv, sh], axis=1)
    zr = jnp.dot(cat, wzr_ref[...], preferred_element_type=f32) + bzr_ref[...]
    z = jax.nn.sigmoid(zr[:, :D])
    r = jax.nn.sigmoid(zr[:, D:])
    g = hv * r
    p = (jnp.dot(jnp.concatenate([xv, sx], axis=1), wxh2_ref[...],
                 preferred_element_type=f32)
         + jnp.dot(g, whh0_ref[...], preferred_element_type=f32)
         + bh_ref[...])
    z_ref[...] = z
    p_ref[...] = p
    gs_ref[...] = -(g * dis)


def _stage_d(x, H, sxr, shr, disb, wzr, bzr, wxh2, whh0, bh):
    return pl.pallas_call(
        _gates_body,
        grid=(GRID,),
        in_specs=[_row_spec()] * 5 + [
            _full_spec((4 * D, 2 * D)), _full_spec((1, 2 * D)),
            _full_spec((2 * D, D)), _full_spec((D, D)), _full_spec((1, D)),
        ],
        out_specs=[_row_spec()] * 3,
        out_shape=[jax.ShapeDtypeStruct((N, D), f32)] * 3,
    )(x, H, sxr, shr, disb, wzr, bzr, wxh2, whh0, bh)


def _update_body(z_ref, p_ref, sg0_ref, sg1_ref, disb_ref, h_ref,
                 whh1_ref, wsum_ref, hn_ref, hr_ref, hwr_ref):
    sg = (sg0_ref[...] + sg1_ref[...]) * disb_ref[...]
    ht = jnp.tanh(p_ref[...] + jnp.dot(sg, whh1_ref[...],
                                       preferred_element_type=f32))
    z = z_ref[...]
    hn = z * h_ref[...] + (1.0 - z) * ht
    hrelu = jnp.maximum(hn, 0.0)
    hn_ref[...] = hn
    hr_ref[...] = hrelu
    hwr_ref[...] = hrelu * wsum_ref[...]


def _stage_f(z, p, sg0, sg1, disb, H, whh1, wsum):
    return pl.pallas_call(
        _update_body,
        grid=(GRID,),
        in_specs=[_row_spec()] * 6 + [_full_spec((D, D)), _full_spec((1, D))],
        out_specs=[_row_spec()] * 3,
        out_shape=[jax.ShapeDtypeStruct((N, D), f32)] * 3,
    )(z, p, sg0, sg1, disb, H, whh1, wsum)


def kernel(x, edge_index, edge_label_index, H,
           W_xz, b_xz, W_hz, b_hz, W_xr, b_xr, W_hr, b_hr,
           W_xh, b_xh, W_hh, b_hh, W_post, b_post):
    src = edge_index[0]
    dst = edge_index[1]
    srcl = edge_label_index[0]
    dstl = edge_label_index[1]

    zeros128 = jnp.zeros((ROWS_PER_TILE, D), f32)

    ones128 = jnp.ones((CB, D), f32)
    deg0, deg1 = _deg_kernel(src, ones128, zeros128)
    degb = jnp.broadcast_to((deg0[:N, 0] + deg1[:N, 0])[:, None], (N, D))

    xs, hs, disb = _stage_b(x, H, degb)

    sxr, shr = _prop2_kernel(xs, hs, src, dst, zeros128)
    sxr, shr = sxr[:N], shr[:N]

    wzr = jnp.concatenate([
        jnp.concatenate([W_xz[0], W_xr[0]], axis=1),
        jnp.concatenate([W_xz[1], W_xr[1]], axis=1),
        jnp.concatenate([W_hz[0], W_hr[0]], axis=1),
        jnp.concatenate([W_hz[1], W_hr[1]], axis=1),
    ], axis=0)
    bzr = jnp.concatenate([b_xz + b_hz, b_xr + b_hr])[None, :]
    wxh2 = jnp.concatenate([W_xh[0], W_xh[1]], axis=0)
    bh = (b_xh + b_hh)[None, :]
    z, p, gs = _stage_d(x, H, sxr, shr, disb, wzr, bzr, wxh2, W_hh[0], bh)

    sg0, sg1 = _prop1_kernel(gs, src, dst, zeros128)
    sg0, sg1 = sg0[:N], sg1[:N]

    wsum = (W_post[0] + W_post[1])[None, :]
    hn, h, hw = _stage_f(z, p, sg0, sg1, disb, H, W_hh[1], wsum)

    part = _pred_kernel(h, hw, srcl, dstl)

    part2d = part.reshape(EL * 16 // D, D)
    smat = jnp.repeat(jnp.eye(8, dtype=f32), 16, axis=0)
    csum8 = jnp.full((1, 8), b_post[0] + b_post[1], f32)
    pred = _stage_h(part2d, smat, csum8).reshape(EL)

    return (pred, hn)


# speedup vs baseline: 14.0061x; 1.0626x over previous
import functools

import jax
import jax.numpy as jnp
from jax import lax
from jax.experimental import pallas as pl
from jax.experimental.pallas import tpu as pltpu
from jax.experimental.pallas import tpu_sc as plsc

N = 10000
E = 320000
EL = 100000
D = 128

NC = 2
NS = 16
CB = 128
NCHUNKS_E = E // CB
N_PAD = 10240
ROWS_PER_TILE = N_PAD // NS
NB = 6
CBP = 32
NBP = 10
NCHUNKS_P = E // CBP
NBG = 4
CBG = 64

_MESH = plsc.VectorSubcoreMesh(
    core_axis_name="c", subcore_axis_name="s", num_cores=NC, num_subcores=NS)

f32 = jnp.float32
i32 = jnp.int32


@functools.partial(
    pl.kernel,
    out_type=(jax.ShapeDtypeStruct((N_PAD, D), f32),
              jax.ShapeDtypeStruct((N_PAD, D), f32)),
    mesh=_MESH,
    scratch_types=[
        pltpu.VMEM((NB, CB), i32),
        pltpu.VMEM((CB, D), f32),
        pltpu.VMEM_SHARED((N_PAD, D), f32),
    ] + [pltpu.SemaphoreType.DMA] * (2 * NB),
)
def _deg_kernel(src_hbm, ones_hbm, zeros_hbm, deg0_hbm, deg1_hbm,
                idx_v, ones_v, acc_sh, *sems):
    sem_i = sems[0:NB]
    sem_s = sems[NB:2 * NB]
    c = lax.axis_index("c")
    s = lax.axis_index("s")
    wid = c * NS + s
    stride = NC * NS

    pltpu.sync_copy(zeros_hbm.at[pl.ds(0, ROWS_PER_TILE)],
                    acc_sh.at[pl.ds(s * ROWS_PER_TILE, ROWS_PER_TILE)])
    pltpu.sync_copy(ones_hbm, ones_v)
    plsc.subcore_barrier()

    cnt = (NCHUNKS_E - wid + (stride - 1)) // stride
    nsup = cnt // NB

    def body(i, _):
        descs = []
        for k in range(NB):
            base = (wid + (i * NB + k) * stride) * CB
            descs.append(pltpu.async_copy(src_hbm.at[pl.ds(base, CB)],
                                          idx_v.at[k], sem_i[k]))
        sd = []
        for k in range(NB):
            descs[k].wait()
            sd.append(pltpu.async_copy(ones_v, acc_sh.at[idx_v.at[k]],
                                       sem_s[k], add=True))
        for k in range(NB):
            sd[k].wait()
        return 0
    lax.fori_loop(0, nsup, body, 0)

    def tail(j, _):
        base = (wid + (nsup * NB + j) * stride) * CB
        pltpu.sync_copy(src_hbm.at[pl.ds(base, CB)], idx_v.at[0])
        pltpu.sync_copy(ones_v, acc_sh.at[idx_v.at[0]], add=True)
        return 0
    lax.fori_loop(0, cnt - nsup * NB, tail, 0)
    plsc.subcore_barrier()

    @pl.when(c == 0)
    def _():
        pltpu.sync_copy(acc_sh.at[pl.ds(s * ROWS_PER_TILE, ROWS_PER_TILE)],
                        deg0_hbm.at[pl.ds(s * ROWS_PER_TILE, ROWS_PER_TILE)])

    @pl.when(c == 1)
    def _():
        pltpu.sync_copy(acc_sh.at[pl.ds(s * ROWS_PER_TILE, ROWS_PER_TILE)],
                        deg1_hbm.at[pl.ds(s * ROWS_PER_TILE, ROWS_PER_TILE)])


@functools.partial(
    pl.kernel,
    out_type=(jax.ShapeDtypeStruct((N_PAD, D), f32),
              jax.ShapeDtypeStruct((N_PAD, D), f32)),
    mesh=_MESH,
    scratch_types=[
        pltpu.VMEM((NBP, CBP), i32),
        pltpu.VMEM((NBP, CBP), i32),
        pltpu.VMEM((NBP, CBP, D), f32),
        pltpu.VMEM_SHARED((N_PAD, D), f32),
    ] + [pltpu.SemaphoreType.DMA] * (3 * NBP),
)
def _prop2_kernel(xs_hbm, hs_hbm, src_hbm, dst_hbm, zeros_hbm,
                  outx_hbm, outh_hbm, idxs_v, idxd_v, rows_v, acc_sh, *sems):
    sem_i = sems[0:NBP]
    sem_g = sems[NBP:2 * NBP]
    sem_s = sems[2 * NBP:3 * NBP]
    c = lax.axis_index("c")
    s = lax.axis_index("s")

    pltpu.sync_copy(zeros_hbm,
                    acc_sh.at[pl.ds(s * ROWS_PER_TILE, ROWS_PER_TILE)])
    plsc.subcore_barrier()

    def run(table_hbm):
        cnt = (NCHUNKS_P - s + (NS - 1)) // NS
        nsup = cnt // NBP

        def body(i, _):
            descs = []
            for k in range(NBP):
                base = (s + (i * NBP + k) * NS) * CBP
                di = pltpu.async_copy(src_hbm.at[pl.ds(base, CBP)],
                                      idxs_v.at[k], sem_i[k])
                dj = pltpu.async_copy(dst_hbm.at[pl.ds(base, CBP)],
                                      idxd_v.at[k], sem_i[k])
                descs.append((di, dj))
            gd = []
            for k in range(NBP):
                descs[k][0].wait()
                descs[k][1].wait()
                gd.append(pltpu.async_copy(table_hbm.at[idxs_v.at[k]],
                                           rows_v.at[k], sem_g[k]))
            sd = []
            for k in range(NBP):
                gd[k].wait()
                sd.append(pltpu.async_copy(rows_v.at[k],
                                           acc_sh.at[idxd_v.at[k]],
                                           sem_s[k], add=True))
            for k in range(NBP):
                sd[k].wait()
            return 0
        lax.fori_loop(0, nsup, body, 0)

        def tail(j, _):
            base = (s + (nsup * NBP + j) * NS) * CBP
            pltpu.sync_copy(src_hbm.at[pl.ds(base, CBP)], idxs_v.at[0])
            pltpu.sync_copy(dst_hbm.at[pl.ds(base, CBP)], idxd_v.at[0])
            pltpu.async_copy(table_hbm.at[idxs_v.at[0]],
                             rows_v.at[0], sem_g[0]).wait()
            pltpu.sync_copy(rows_v.at[0], acc_sh.at[idxd_v.at[0]], add=True)
            return 0
        lax.fori_loop(0, cnt - nsup * NBP, tail, 0)

    @pl.when(c == 0)
    def _():
        run(xs_hbm)

    @pl.when(c == 1)
    def _():
        run(hs_hbm)

    plsc.subcore_barrier()

    @pl.when(c == 0)
    def _():
        pltpu.sync_copy(acc_sh.at[pl.ds(s * ROWS_PER_TILE, ROWS_PER_TILE)],
                        outx_hbm.at[pl.ds(s * ROWS_PER_TILE, ROWS_PER_TILE)])

    @pl.when(c == 1)
    def _():
        pltpu.sync_copy(acc_sh.at[pl.ds(s * ROWS_PER_TILE, ROWS_PER_TILE)],
                        outh_hbm.at[pl.ds(s * ROWS_PER_TILE, ROWS_PER_TILE)])


@functools.partial(
    pl.kernel,
    out_type=(jax.ShapeDtypeStruct((N_PAD, D), f32),
              jax.ShapeDtypeStruct((N_PAD, D), f32)),
    mesh=_MESH,
    scratch_types=[
        pltpu.VMEM((NBP, CBP), i32),
        pltpu.VMEM((NBP, CBP), i32),
        pltpu.VMEM((NBP, CBP, D), f32),
        pltpu.VMEM_SHARED((N_PAD, D), f32),
    ] + [pltpu.SemaphoreType.DMA] * (3 * NBP),
)
def _prop1_kernel(gs_hbm, src_hbm, dst_hbm, zeros_hbm,
                  out0_hbm, out1_hbm, idxs_v, idxd_v, rows_v, acc_sh, *sems):
    sem_i = sems[0:NBP]
    sem_g = sems[NBP:2 * NBP]
    sem_s = sems[2 * NBP:3 * NBP]
    c = lax.axis_index("c")
    s = lax.axis_index("s")
    wid = c * NS + s
    stride = NC * NS

    pltpu.sync_copy(zeros_hbm,
                    acc_sh.at[pl.ds(s * ROWS_PER_TILE, ROWS_PER_TILE)])
    plsc.subcore_barrier()

    cnt = (NCHUNKS_P - wid + (stride - 1)) // stride
    nsup = cnt // NBP

    def body(i, _):
        descs = []
        for k in range(NBP):
            base = (wid + (i * NBP + k) * stride) * CBP
            di = pltpu.async_copy(src_hbm.at[pl.ds(base, CBP)],
                                  idxs_v.at[k], sem_i[k])
            dj = pltpu.async_copy(dst_hbm.at[pl.ds(base, CBP)],
                                  idxd_v.at[k], sem_i[k])
            descs.append((di, dj))
        gd = []
        for k in range(NBP):
            descs[k][0].wait()
            descs[k][1].wait()
            gd.append(pltpu.async_copy(gs_hbm.at[idxs_v.at[k]],
                                       rows_v.at[k], sem_g[k]))
        sd = []
        for k in range(NBP):
            gd[k].wait()
            sd.append(pltpu.async_copy(rows_v.at[k],
                                       acc_sh.at[idxd_v.at[k]],
                                       sem_s[k], add=True))
        for k in range(NBP):
            sd[k].wait()
        return 0
    lax.fori_loop(0, nsup, body, 0)

    def tail(j, _):
        base = (wid + (nsup * NBP + j) * stride) * CBP
        pltpu.sync_copy(src_hbm.at[pl.ds(base, CBP)], idxs_v.at[0])
        pltpu.sync_copy(dst_hbm.at[pl.ds(base, CBP)], idxd_v.at[0])
        pltpu.async_copy(gs_hbm.at[idxs_v.at[0]],
                         rows_v.at[0], sem_g[0]).wait()
        pltpu.sync_copy(rows_v.at[0], acc_sh.at[idxd_v.at[0]], add=True)
        return 0
    lax.fori_loop(0, cnt - nsup * NBP, tail, 0)
    plsc.subcore_barrier()

    @pl.when(c == 0)
    def _():
        pltpu.sync_copy(acc_sh.at[pl.ds(s * ROWS_PER_TILE, ROWS_PER_TILE)],
                        out0_hbm.at[pl.ds(s * ROWS_PER_TILE, ROWS_PER_TILE)])

    @pl.when(c == 1)
    def _():
        pltpu.sync_copy(acc_sh.at[pl.ds(s * ROWS_PER_TILE, ROWS_PER_TILE)],
                        out1_hbm.at[pl.ds(s * ROWS_PER_TILE, ROWS_PER_TILE)])


NCHUNKS_L = (EL + CBG - 1) // CBG
LAST_START = EL - CBG


@functools.partial(
    pl.kernel,
    out_type=jax.ShapeDtypeStruct((EL, 16), f32),
    mesh=_MESH,
    scratch_types=[
        pltpu.VMEM((NBG, CBG), i32),
        pltpu.VMEM((NBG, CBG), i32),
        pltpu.VMEM((NBG, CBG, D), f32),
        pltpu.VMEM((NBG, CBG, D), f32),
        pltpu.VMEM((NBG, CBG, 16), f32),
    ] + [pltpu.SemaphoreType.DMA] * (3 * NBG),
)
def _pred_kernel(h_hbm, hw_hbm, srcl_hbm, dstl_hbm, part_hbm,
                 idxa_v, idxb_v, ra_v, rb_v, part_v, *sems):
    sem_i = sems[0:NBG]
    sem_g = sems[NBG:2 * NBG]
    sem_w = sems[2 * NBG:3 * NBG]
    c = lax.axis_index("c")
    s = lax.axis_index("s")
    wid = c * NS + s
    stride = NC * NS

    cnt = (NCHUNKS_L - wid + (stride - 1)) // stride
    nsup = cnt // NBG

    def compute(k, start):
        def ebody(e, _):
            acc = ra_v[k, e, pl.ds(0, 16)] * rb_v[k, e, pl.ds(0, 16)]
            for q in range(1, D // 16):
                acc = acc + (ra_v[k, e, pl.ds(q * 16, 16)]
                             * rb_v[k, e, pl.ds(q * 16, 16)])
            part_v[k, e, :] = acc
            return 0
        lax.fori_loop(0, CBG, ebody, 0)
        return pltpu.async_copy(part_v.at[k], part_hbm.at[pl.ds(start, CBG)],
                                sem_w[k])

    def body(i, _):
        starts = []
        descs = []
        for k in range(NBG):
            chunk = wid + (i * NBG + k) * stride
            start = pl.multiple_of(jnp.minimum(chunk * CBG, LAST_START), 32)
            starts.append(start)
            da = pltpu.async_copy(srcl_hbm.at[pl.ds(start, CBG)],
                                  idxa_v.at[k], sem_i[k])
            db = pltpu.async_copy(dstl_hbm.at[pl.ds(start, CBG)],
                                  idxb_v.at[k], sem_i[k])
            descs.append((da, db))
        gd = []
        for k in range(NBG):
            descs[k][0].wait()
            descs[k][1].wait()
            ga = pltpu.async_copy(h_hbm.at[idxa_v.at[k]], ra_v.at[k],
                                  sem_g[k])
            gb = pltpu.async_copy(hw_hbm.at[idxb_v.at[k]], rb_v.at[k],
                                  sem_g[k])
            gd.append((ga, gb))
        wd = []
        for k in range(NBG):
            gd[k][0].wait()
            gd[k][1].wait()
            wd.append(compute(k, starts[k]))
        for k in range(NBG):
            wd[k].wait()
        return 0
    lax.fori_loop(0, nsup, body, 0)

    def tail(j, _):
        chunk = wid + (nsup * NBG + j) * stride
        start = pl.multiple_of(jnp.minimum(chunk * CBG, LAST_START), 32)
        pltpu.sync_copy(srcl_hbm.at[pl.ds(start, CBG)], idxa_v.at[0])
        pltpu.sync_copy(dstl_hbm.at[pl.ds(start, CBG)], idxb_v.at[0])
        pltpu.async_copy(h_hbm.at[idxa_v.at[0]], ra_v.at[0], sem_g[0]).wait()
        pltpu.async_copy(hw_hbm.at[idxb_v.at[0]], rb_v.at[0], sem_g[0]).wait()
        compute(0, start).wait()
        return 0
    lax.fori_loop(0, cnt - nsup * NBG, tail, 0)


def _reduce_body(p_ref, s_ref, c_ref, out_ref):
    out_ref[...] = (jnp.dot(p_ref[...], s_ref[...], preferred_element_type=f32)
                    + c_ref[...])


def _stage_h(part2d, smat, csum8):
    nrows = EL * 16 // D
    return pl.pallas_call(
        _reduce_body,
        out_shape=jax.ShapeDtypeStruct((nrows, 8), f32),
    )(part2d, smat, csum8)


RB = 1000
GRID = N // RB


def _row_spec():
    return pl.BlockSpec((RB, D), lambda i: (i, 0))


def _full_spec(shape):
    return pl.BlockSpec(shape, lambda i: tuple(0 for _ in shape))


def _scale_body(x_ref, h_ref, degb_ref, xs_ref, hs_ref, disb_ref):
    deg = degb_ref[...]
    dis = jnp.where(deg > 0, lax.rsqrt(jnp.maximum(deg, 1e-12)), 0.0)
    xs_ref[...] = -(x_ref[...] * dis)
    hs_ref[...] = -(h_ref[...] * dis)
    disb_ref[...] = dis


def _stage_b(x, H, degb):
    return pl.pallas_call(
        _scale_body,
        grid=(GRID,),
        in_specs=[_row_spec()] * 3,
        out_specs=[_row_spec()] * 3,
        out_shape=[jax.ShapeDtypeStruct((N, D), f32)] * 3,
    )(x, H, degb)


def _gates_body(x_ref, h_ref, sxr_ref, shr_ref, disb_ref,
                wzr_ref, bzr_ref, wxh2_ref, whh0_ref, bh_ref,
                z_ref, p_ref, gs_ref):
    dis = disb_ref[...]
    xv = x_ref[...]
    hv = h_ref[...]
    sx = sxr_ref[...] * dis
    sh = shr_ref[...] * dis
    cat = jnp.concatenate([xv, sx, hv, sh], axis=1)
    zr = jnp.dot(cat, wzr_ref[...], preferred_element_type=f32) + bzr_ref[...]
    z = jax.nn.sigmoid(zr[:, :D])
    r = jax.nn.sigmoid(zr[:, D:])
    g = hv * r
    p = (jnp.dot(jnp.concatenate([xv, sx], axis=1), wxh2_ref[...],
                 preferred_element_type=f32)
         + jnp.dot(g, whh0_ref[...], preferred_element_type=f32)
         + bh_ref[...])
    z_ref[...] = z
    p_ref[...] = p
    gs_ref[...] = -(g * dis)


def _stage_d(x, H, sxr, shr, disb, wzr, bzr, wxh2, whh0, bh):
    return pl.pallas_call(
        _gates_body,
        grid=(GRID,),
        in_specs=[_row_spec()] * 5 + [
            _full_spec((4 * D, 2 * D)), _full_spec((1, 2 * D)),
            _full_spec((2 * D, D)), _full_spec((D, D)), _full_spec((1, D)),
        ],
        out_specs=[_row_spec()] * 3,
        out_shape=[jax.ShapeDtypeStruct((N, D), f32)] * 3,
    )(x, H, sxr, shr, disb, wzr, bzr, wxh2, whh0, bh)


def _update_body(z_ref, p_ref, sg0_ref, sg1_ref, disb_ref, h_ref,
                 whh1_ref, wsum_ref, hn_ref, hr_ref, hwr_ref):
    sg = (sg0_ref[...] + sg1_ref[...]) * disb_ref[...]
    ht = jnp.tanh(p_ref[...] + jnp.dot(sg, whh1_ref[...],
                                       preferred_element_type=f32))
    z = z_ref[...]
    hn = z * h_ref[...] + (1.0 - z) * ht
    hrelu = jnp.maximum(hn, 0.0)
    hn_ref[...] = hn
    hr_ref[...] = hrelu
    hwr_ref[...] = hrelu * wsum_ref[...]


def _stage_f(z, p, sg0, sg1, disb, H, whh1, wsum):
    return pl.pallas_call(
        _update_body,
        grid=(GRID,),
        in_specs=[_row_spec()] * 6 + [_full_spec((D, D)), _full_spec((1, D))],
        out_specs=[_row_spec()] * 3,
        out_shape=[jax.ShapeDtypeStruct((N, D), f32)] * 3,
    )(z, p, sg0, sg1, disb, H, whh1, wsum)


def kernel(x, edge_index, edge_label_index, H,
           W_xz, b_xz, W_hz, b_hz, W_xr, b_xr, W_hr, b_hr,
           W_xh, b_xh, W_hh, b_hh, W_post, b_post):
    src = edge_index[0]
    dst = edge_index[1]
    srcl = edge_label_index[0]
    dstl = edge_label_index[1]

    zeros128 = jnp.zeros((ROWS_PER_TILE, D), f32)

    ones128 = jnp.ones((CB, D), f32)
    deg0, deg1 = _deg_kernel(src, ones128, zeros128)
    degb = jnp.broadcast_to((deg0[:N, 0] + deg1[:N, 0])[:, None], (N, D))

    xs, hs, disb = _stage_b(x, H, degb)

    sxr, shr = _prop2_kernel(xs, hs, src, dst, zeros128)
    sxr, shr = sxr[:N], shr[:N]

    wzr = jnp.concatenate([
        jnp.concatenate([W_xz[0], W_xr[0]], axis=1),
        jnp.concatenate([W_xz[1], W_xr[1]], axis=1),
        jnp.concatenate([W_hz[0], W_hr[0]], axis=1),
        jnp.concatenate([W_hz[1], W_hr[1]], axis=1),
    ], axis=0)
    bzr = jnp.concatenate([b_xz + b_hz, b_xr + b_hr])[None, :]
    wxh2 = jnp.concatenate([W_xh[0], W_xh[1]], axis=0)
    bh = (b_xh + b_hh)[None, :]
    z, p, gs = _stage_d(x, H, sxr, shr, disb, wzr, bzr, wxh2, W_hh[0], bh)

    sg0, sg1 = _prop1_kernel(gs, src, dst, zeros128)
    sg0, sg1 = sg0[:N], sg1[:N]

    wsum = (W_post[0] + W_post[1])[None, :]
    hn, h, hw = _stage_f(z, p, sg0, sg1, disb, H, W_hh[1], wsum)

    part = _pred_kernel(h, hw, srcl, dstl)

    part2d = part.reshape(EL * 16 // D, D)
    smat = jnp.repeat(jnp.eye(8, dtype=f32), 16, axis=0)
    csum8 = jnp.full((1, 8), b_post[0] + b_post[1], f32)
    pred = _stage_h(part2d, smat, csum8).reshape(EL)

    return (pred, hn)
